# baseline (speedup 1.0000x reference)
import jax
import jax.numpy as jnp
from jax import lax
from jax.experimental import pallas as pl
from jax.experimental.pallas import tpu as pltpu
from jax.experimental.pallas import tpu_sc as plsc

B = 16
V = 4096
NUM_GENES = 20000
LATENT = 64

NC = 2
NS = 16
L = 16
NW = NC * NS
GPT = NUM_GENES // NW
TBL = GPT * LATENT
CHUNK = 128
NVEC = V // L
UNROLL = 4


def _sc_kernel(emb_hbm, gid_hbm, mask_hbm, out_hbm,
               ids_v, mask_v, cidx_v, clg_v, rows_v, table_v, gsem, osem0, osem1, ssem):
    wid = lax.axis_index("s") * NC + lax.axis_index("c")
    lo = wid * GPT
    out_base = wid * TBL

    zeros16 = jnp.zeros((L,), jnp.float32)
    neginf16 = jnp.full((L,), -jnp.inf, jnp.float32)
    iota16 = lax.iota(jnp.int32, L)
    ugpt = jnp.uint32(GPT)

    pltpu.make_async_copy(gid_hbm.at[0], ids_v.at[0], ssem).start()
    pltpu.make_async_copy(mask_hbm.at[0], mask_v.at[0, pl.ds(0, V)], ssem).start()

    def _zf(i, _):
        for k in range(10):
            table_v[0, pl.ds((i * 10 + k) * L, L)] = zeros16
            table_v[1, pl.ds((i * 10 + k) * L, L)] = zeros16
        return _
    lax.fori_loop(0, TBL // L // 10, _zf, None)

    osems = (osem0, osem1)
    cnt_prev = [None, None]

    for b in range(B):
        p = b & 1
        cidx_p = cidx_v.at[p]
        clg_p = clg_v.at[p]
        table_p = table_v.at[p]

        ids_p = ids_v.at[p]
        mask_p = mask_v.at[p]
        with jax.named_scope("stage_ids"):
            pltpu.make_async_copy(gid_hbm.at[b], ids_p, ssem).wait()
            pltpu.make_async_copy(mask_hbm.at[b], mask_v.at[p, pl.ds(0, V)], ssem).wait()
            if b + 1 < B:
                pltpu.make_async_copy(gid_hbm.at[b + 1], ids_v.at[1 - p], ssem).start()
                pltpu.make_async_copy(
                    mask_hbm.at[b + 1], mask_v.at[1 - p, pl.ds(0, V)], ssem).start()

        if b >= 2:
          with jax.named_scope("drain_restore"):
            pltpu.make_async_copy(
                table_p,
                out_hbm.at[pl.ds((b - 2) * NUM_GENES * LATENT + out_base, TBL)],
                osems[p]).wait()

            def _restore(j, _):
                tb = clg_p[pl.ds(j, L)][0] * LATENT
                for k in range(LATENT // L):
                    table_p[pl.ds(tb + k * L, L)] = zeros16
                return _
            lax.fori_loop(0, cnt_prev[p], _restore, None)

        base0 = b * V + iota16

        def _scan(i, cnt):
            for u in range(UNROLL):
                vi = i * UNROLL + u
                ids = ids_p[pl.ds(vi * L, L)]
                rel = ids - lo
                m = rel.astype(jnp.uint32) < ugpt
                pos = cnt + plsc.cumsum(m.astype(jnp.int32)) - 1
                plsc.store_scatter(cidx_p, [pos], base0 + vi * L, mask=m)
                plsc.store_scatter(clg_p, [pos], rel, mask=m)
                cnt = pos[15] + 1
            return cnt
        with jax.named_scope("scan"):
            cnt = lax.fori_loop(0, NVEC // UNROLL, _scan, 0)

        zi = jnp.zeros((L,), jnp.int32)
        for k in range(CHUNK // L):
            cidx_p[pl.ds(cnt + k * L, L)] = zi

        nch = (cnt + CHUNK - 1) // CHUNK

        def _fire(c):
            def _f(j, _):
                gi = cidx_p[pl.ds(c * CHUNK + j, L)][0]
                pltpu.make_async_copy(
                    emb_hbm.at[gi], rows_v.at[j], gsem).start()
                return _
            lax.fori_loop(0, CHUNK, _f, None)
        _fire(0)

        def _prime(j, _):
            tb = clg_p[pl.ds(j, L)][0] * LATENT
            for k in range(LATENT // L):
                table_p[pl.ds(tb + k * L, L)] = neginf16
            return _
        with jax.named_scope("prime"):
            lax.fori_loop(0, cnt, _prime, None)

        def _chunk(c, _):
            with jax.named_scope("gwait"):
                pltpu.make_async_copy(emb_hbm.at[pl.ds(0, CHUNK)], rows_v, gsem).wait()
            nrows = jnp.minimum(CHUNK, cnt - c * CHUNK)

            def _row(j, _):
                gi = cidx_p[pl.ds(c * CHUNK + j, L)][0]
                mval = mask_p[pl.ds(gi - b * V, L)][0]
                tb = clg_p[pl.ds(c * CHUNK + j, L)][0] * LATENT
                for k in range(LATENT // L):
                    row = rows_v[j, pl.ds(k * L, L)] * mval
                    cur = table_p[pl.ds(tb + k * L, L)]
                    table_p[pl.ds(tb + k * L, L)] = jnp.maximum(cur, row)
                return _
            with jax.named_scope("rows"):
                lax.fori_loop(0, nrows, _row, None)

            @pl.when(c + 1 < nch)
            def _fire_next():
                _fire(c + 1)
            return _
        with jax.named_scope("chunks"):
            lax.fori_loop(0, nch, _chunk, None)

        pltpu.make_async_copy(
            table_p,
            out_hbm.at[pl.ds(b * NUM_GENES * LATENT + out_base, TBL)],
            osems[p]).start()
        cnt_prev[p] = cnt

    for b in (B - 2, B - 1):
        p = b & 1
        pltpu.make_async_copy(
            table_v.at[p],
            out_hbm.at[pl.ds(b * NUM_GENES * LATENT + out_base, TBL)],
            osems[p]).wait()


@jax.jit
def kernel(variant_embeddings, gene_ids, mask):
    emb2d = variant_embeddings.reshape(B * V, LATENT)
    maskf = mask.astype(jnp.float32)
    mesh = plsc.VectorSubcoreMesh(
        core_axis_name="c", subcore_axis_name="s", num_cores=NC, num_subcores=NS)
    out = pl.kernel(
        _sc_kernel,
        out_type=jax.ShapeDtypeStruct((B * NUM_GENES * LATENT,), jnp.float32),
        mesh=mesh,
        compiler_params=pltpu.CompilerParams(
            needs_layout_passes=False, use_tc_tiling_on_sc=False),
        scratch_types=[
            pltpu.VMEM((2, V), jnp.int32),
            pltpu.VMEM((2, V + L), jnp.float32),
            pltpu.VMEM((2, V + CHUNK), jnp.int32),
            pltpu.VMEM((2, V + CHUNK), jnp.int32),
            pltpu.VMEM((CHUNK, LATENT), jnp.float32),
            pltpu.VMEM((2, TBL), jnp.float32),
            pltpu.SemaphoreType.DMA,
            pltpu.SemaphoreType.DMA,
            pltpu.SemaphoreType.DMA,
            pltpu.SemaphoreType.DMA,
        ],
    )(emb2d, gene_ids, maskf)
    return out.reshape(B, NUM_GENES, LATENT)


# speedup vs baseline: 2.0690x; 2.0690x over previous
import jax
import jax.numpy as jnp
from jax import lax
from jax.experimental import pallas as pl
from jax.experimental.pallas import tpu as pltpu
from jax.experimental.pallas import tpu_sc as plsc

B = 16
V = 4096
NUM_GENES = 20000
LATENT = 64

NC = 2
NS = 16
L = 16
NW = NC * NS
GPT = NUM_GENES // NW
TBL = GPT * LATENT
CHUNK = 128
NVEC = V // L
UNROLL = 4
VS = V // NS


def _sc_kernel(emb_hbm, gid_hbm, mask_hbm, out_hbm,
               ids_v, mask_v, cidx_v, clg_v, rows_v, table_v, spm_v,
               gsem, osem, ssem, stsem):
    wid = lax.axis_index("s") * NC + lax.axis_index("c")
    sid = lax.axis_index("s")
    lo = wid * GPT
    out_base = wid * TBL

    zeros16 = jnp.zeros((L,), jnp.float32)
    neginf16 = jnp.full((L,), -jnp.inf, jnp.float32)
    iota16 = lax.iota(jnp.int32, L)
    ugpt = jnp.uint32(GPT)

    pltpu.make_async_copy(gid_hbm.at[0], ids_v.at[0], ssem).start()
    pltpu.make_async_copy(mask_hbm.at[0], mask_v.at[0, pl.ds(0, V)], ssem).start()

    def _zf(i, _):
        for k in range(10):
            table_v[pl.ds((i * 10 + k) * L, L)] = zeros16
        return _
    lax.fori_loop(0, TBL // L // 10, _zf, None)

    cnt_prev = None

    for b in range(B):
        p = b & 1
        cidx_p = cidx_v.at[p]
        clg_p = clg_v.at[p]

        plsc.subcore_barrier()
        stg = pltpu.make_async_copy(
            emb_hbm.at[pl.ds(b * V + sid * VS, VS)],
            spm_v.at[pl.ds(sid * VS, VS)], stsem)
        stg.start()

        ids_p = ids_v.at[p]
        mask_p = mask_v.at[p]
        with jax.named_scope("stage_ids"):
            pltpu.make_async_copy(gid_hbm.at[b], ids_p, ssem).wait()
            pltpu.make_async_copy(mask_hbm.at[b], mask_v.at[p, pl.ds(0, V)], ssem).wait()
            if b + 1 < B:
                pltpu.make_async_copy(gid_hbm.at[b + 1], ids_v.at[1 - p], ssem).start()
                pltpu.make_async_copy(
                    mask_hbm.at[b + 1], mask_v.at[1 - p, pl.ds(0, V)], ssem).start()

        if b >= 1:
            with jax.named_scope("drain_restore"):
                pltpu.make_async_copy(
                    table_v,
                    out_hbm.at[pl.ds((b - 1) * NUM_GENES * LATENT + out_base, TBL)],
                    osem).wait()

                clg_q = clg_v.at[1 - p]

                def _restore(j, _):
                    tb = clg_q[pl.ds(j, L)][0] * LATENT
                    for k in range(LATENT // L):
                        table_v[pl.ds(tb + k * L, L)] = zeros16
                    return _
                lax.fori_loop(0, cnt_prev, _restore, None)

        def _scan(i, cnt):
            for u in range(UNROLL):
                vi = i * UNROLL + u
                ids = ids_p[pl.ds(vi * L, L)]
                rel = ids - lo
                m = rel.astype(jnp.uint32) < ugpt
                pos = cnt + plsc.cumsum(m.astype(jnp.int32)) - 1
                plsc.store_scatter(cidx_p, [pos], iota16 + vi * L, mask=m)
                plsc.store_scatter(clg_p, [pos], rel, mask=m)
                cnt = pos[15] + 1
            return cnt
        with jax.named_scope("scan"):
            cnt = lax.fori_loop(0, NVEC // UNROLL, _scan, 0)

        zi = jnp.zeros((L,), jnp.int32)
        for k in range(CHUNK // L):
            cidx_p[pl.ds(cnt + k * L, L)] = zi

        nch = (cnt + CHUNK - 1) // CHUNK

        def _prime(j, _):
            tb = clg_p[pl.ds(j, L)][0] * LATENT
            for k in range(LATENT // L):
                table_v[pl.ds(tb + k * L, L)] = neginf16
            return _
        with jax.named_scope("prime"):
            lax.fori_loop(0, cnt, _prime, None)

        with jax.named_scope("stwait"):
            stg.wait()
            plsc.subcore_barrier()

        pltpu.make_async_copy(
            spm_v.at[cidx_p.at[pl.ds(0, CHUNK)]], rows_v, gsem).start()

        def _chunk(c, _):
            with jax.named_scope("gwait"):
                pltpu.make_async_copy(
                    spm_v.at[cidx_p.at[pl.ds(c * CHUNK, CHUNK)]], rows_v, gsem).wait()
            nrows = jnp.minimum(CHUNK, cnt - c * CHUNK)

            def _row(j, _):
                gi = cidx_p[pl.ds(c * CHUNK + j, L)][0]
                mval = mask_p[pl.ds(gi, L)][0]
                tb = clg_p[pl.ds(c * CHUNK + j, L)][0] * LATENT
                for k in range(LATENT // L):
                    row = rows_v[j, pl.ds(k * L, L)] * mval
                    cur = table_v[pl.ds(tb + k * L, L)]
                    table_v[pl.ds(tb + k * L, L)] = jnp.maximum(cur, row)
                return _
            with jax.named_scope("rows"):
                lax.fori_loop(0, nrows, _row, None)

            @pl.when(c + 1 < nch)
            def _fire_next():
                pltpu.make_async_copy(
                    spm_v.at[cidx_p.at[pl.ds((c + 1) * CHUNK, CHUNK)]],
                    rows_v, gsem).start()
            return _
        with jax.named_scope("chunks"):
            lax.fori_loop(0, nch, _chunk, None)

        pltpu.make_async_copy(
            table_v,
            out_hbm.at[pl.ds(b * NUM_GENES * LATENT + out_base, TBL)],
            osem).start()
        cnt_prev = cnt

    pltpu.make_async_copy(
        table_v,
        out_hbm.at[pl.ds((B - 1) * NUM_GENES * LATENT + out_base, TBL)],
        osem).wait()


@jax.jit
def kernel(variant_embeddings, gene_ids, mask):
    emb2d = variant_embeddings.reshape(B * V, LATENT)
    maskf = mask.astype(jnp.float32)
    mesh = plsc.VectorSubcoreMesh(
        core_axis_name="c", subcore_axis_name="s", num_cores=NC, num_subcores=NS)
    out = pl.kernel(
        _sc_kernel,
        out_type=jax.ShapeDtypeStruct((B * NUM_GENES * LATENT,), jnp.float32),
        mesh=mesh,
        compiler_params=pltpu.CompilerParams(
            needs_layout_passes=False, use_tc_tiling_on_sc=False),
        scratch_types=[
            pltpu.VMEM((2, V), jnp.int32),
            pltpu.VMEM((2, V + L), jnp.float32),
            pltpu.VMEM((2, V + CHUNK), jnp.int32),
            pltpu.VMEM((2, V + CHUNK), jnp.int32),
            pltpu.VMEM((CHUNK, LATENT), jnp.float32),
            pltpu.VMEM((TBL,), jnp.float32),
            pltpu.VMEM_SHARED((V, LATENT), jnp.float32),
            pltpu.SemaphoreType.DMA,
            pltpu.SemaphoreType.DMA,
            pltpu.SemaphoreType.DMA,
            pltpu.SemaphoreType.DMA,
        ],
    )(emb2d, gene_ids, maskf)
    return out.reshape(B, NUM_GENES, LATENT)
